# baseline (speedup 1.0000x reference)
import functools

import jax
import jax.numpy as jnp
from jax import lax
from jax.experimental import pallas as pl
from jax.experimental.pallas import tpu as pltpu
from jax.experimental.pallas import tpu_sc as plsc

_CH = 128


@functools.lru_cache(maxsize=None)
def _build(N, D, NC, NS):
    NW = NC * NS
    per_w = N // NW
    n_ch = per_w // _CH
    mesh = plsc.VectorSubcoreMesh(core_axis_name="c", subcore_axis_name="s")

    @functools.partial(
        pl.kernel,
        mesh=mesh,
        out_type=jax.ShapeDtypeStruct((N, D), jnp.float32),
        scratch_types=[
            pltpu.VMEM((n_ch, _CH), jnp.int32),
            pltpu.VMEM((_CH, D), jnp.float32),
            pltpu.SemaphoreType.DMA,
        ],
        compiler_params=pltpu.CompilerParams(use_tc_tiling_on_sc=False),
    )
    def k(x_hbm, table_hbm, out_hbm, idx_v, rows_v, gsem):
        wid = lax.axis_index("s") * NC + lax.axis_index("c")
        base = wid * per_w
        pltpu.sync_copy(x_hbm.at[wid], idx_v)

        def body(j, carry):
            pltpu.async_copy(table_hbm.at[idx_v.at[j]], rows_v, gsem).wait()
            pltpu.sync_copy(rows_v, out_hbm.at[pl.ds(base + j * _CH, _CH)])
            return carry

        lax.fori_loop(0, n_ch, body, 0)

    return k


def kernel(x, table):
    B, S = x.shape
    V, D = table.shape
    N = B * S
    info = plsc.get_sparse_core_info()
    NC, NS = info.num_cores, info.num_subcores
    NW = NC * NS
    grain = NW * _CH
    Np = ((N + grain - 1) // grain) * grain
    xf = x.reshape(-1).astype(jnp.int32)
    if Np != N:
        xf = jnp.concatenate([xf, jnp.zeros((Np - N,), jnp.int32)])
    xf = xf.reshape(NW, Np // (NW * _CH), _CH)
    out = _build(Np, D, NC, NS)(xf, table)
    if Np != N:
        out = out[:N]
    return out.reshape(B, S, D)


# speedup vs baseline: 1.1106x; 1.1106x over previous
import functools

import jax
import jax.numpy as jnp
from jax import lax
from jax.experimental import pallas as pl
from jax.experimental.pallas import tpu as pltpu
from jax.experimental.pallas import tpu_sc as plsc

_CH = 128
_KC = 4


@functools.lru_cache(maxsize=None)
def _build(N, D, NC, NS):
    NW = NC * NS
    per_w = N // NW
    n_ch = per_w // _CH
    G = n_ch // _KC
    mesh = plsc.VectorSubcoreMesh(core_axis_name="c", subcore_axis_name="s")

    @functools.partial(
        pl.kernel,
        mesh=mesh,
        out_type=jax.ShapeDtypeStruct((N // _CH, _CH, D), jnp.float32),
        scratch_types=[
            pltpu.VMEM((n_ch, _CH), jnp.int32),
            pltpu.VMEM((2, _KC, _CH, D), jnp.float32),
            pltpu.SemaphoreType.DMA,
            pltpu.SemaphoreType.DMA,
        ],
        compiler_params=pltpu.CompilerParams(use_tc_tiling_on_sc=False),
    )
    def k(x_hbm, table_hbm, out_hbm, idx_v, rows_v, sem_a, sem_b):
        wid = lax.axis_index("s") * NC + lax.axis_index("c")
        base = wid * n_ch
        pltpu.sync_copy(x_hbm.at[wid], idx_v)

        def fire(g, b, sem):
            for kk in range(_KC):
                pltpu.async_copy(
                    table_hbm.at[idx_v.at[g * _KC + kk]], rows_v.at[b, kk], sem
                )

        def drain(g, b, sem):
            for kk in range(_KC):
                pltpu.make_async_copy(
                    table_hbm.at[idx_v.at[g * _KC + kk]], rows_v.at[b, kk], sem
                ).wait()

        fire(0, 0, sem_a)

        def step(g, b, sem):
            @pl.when(g + 1 < G)
            def _():
                fire(g + 1, 1 - b, sem_b if b == 0 else sem_a)

            drain(g, b, sem)
            pltpu.sync_copy(
                rows_v.at[b], out_hbm.at[pl.ds(base + g * _KC, _KC)]
            )

        def body(i, carry):
            step(2 * i, 0, sem_a)
            step(2 * i + 1, 1, sem_b)
            return carry

        lax.fori_loop(0, G // 2, body, 0)

    return k


def kernel(x, table):
    B, S = x.shape
    V, D = table.shape
    N = B * S
    info = plsc.get_sparse_core_info()
    NC, NS = info.num_cores, info.num_subcores
    NW = NC * NS
    grain = NW * _CH * _KC * 2
    Np = ((N + grain - 1) // grain) * grain
    xf = x.reshape(-1).astype(jnp.int32)
    if Np != N:
        xf = jnp.concatenate([xf, jnp.zeros((Np - N,), jnp.int32)])
    xf = xf.reshape(NW, Np // (NW * _CH), _CH)
    out = _build(Np, D, NC, NS)(xf, table)
    out = out.reshape(Np, D)
    if Np != N:
        out = out[:N]
    return out.reshape(B, S, D)
